# Initial kernel scaffold; baseline (speedup 1.0000x reference)
import jax
import jax.numpy as jnp
from jax.experimental import pallas as pl


def kernel(x, embeddings):
    raise NotImplementedError("write your pallas kernel here")


# speedup vs baseline: 1.1124x; 1.1124x over previous
import jax
import jax.numpy as jnp
from jax.experimental import pallas as pl
from jax.experimental.pallas import tpu as pltpu
from jax.experimental.pallas import tpu_sc as plsc

_DIM = 64
_NEMB = 1024
_ROWS_PER_BLOCK = 1024
_GATHER_WINDOW = 128


def _distance_argmin_body(x_ref, e_ref, idx_ref, acc_ref):
    i = pl.program_id(0)
    xb = x_ref[...]
    emb = e_ref[...]
    neg2m = jax.lax.dot_general(
        xb * -2.0, emb,
        dimension_numbers=(((1,), (0,)), ((), ())),
        preferred_element_type=jnp.float32,
    )
    x2 = jnp.sum(xb * xb, axis=1, keepdims=True)
    e2 = jnp.sum(emb * emb, axis=0, keepdims=True)
    d = (x2 + e2) + neg2m
    m = jnp.min(d, axis=1, keepdims=True)
    lane = jax.lax.broadcasted_iota(jnp.int32, d.shape, 1)
    idx = jnp.min(jnp.where(d == m, lane, jnp.int32(1 << 30)), axis=1)
    idx_ref[...] = idx.astype(jnp.int32)

    @pl.when(i == 0)
    def _():
        acc_ref[...] = jnp.zeros_like(acc_ref)

    acc_ref[...] += jnp.full(acc_ref.shape, jnp.sum(m), dtype=jnp.float32)


def _distance_argmin(flat_x, embeddings):
    n = flat_x.shape[0]
    nblk = n // _ROWS_PER_BLOCK
    return pl.pallas_call(
        _distance_argmin_body,
        grid=(nblk,),
        in_specs=[
            pl.BlockSpec((_ROWS_PER_BLOCK, _DIM), lambda i: (i, 0)),
            pl.BlockSpec((_DIM, _NEMB), lambda i: (0, 0)),
        ],
        out_specs=[
            pl.BlockSpec((_ROWS_PER_BLOCK,), lambda i: (i,)),
            pl.BlockSpec((8, 128), lambda i: (0, 0)),
        ],
        out_shape=[
            jax.ShapeDtypeStruct((n,), jnp.int32),
            jax.ShapeDtypeStruct((8, 128), jnp.float32),
        ],
        compiler_params=pltpu.CompilerParams(
            dimension_semantics=("arbitrary",)),
    )(flat_x, embeddings)


def _sc_gather(table, indices):
    n = indices.shape[1]
    width = table.shape[1]
    mesh = plsc.VectorSubcoreMesh(
        core_axis_name="core", subcore_axis_name="subcore")

    @pl.kernel(
        out_type=jax.ShapeDtypeStruct((n, width), jnp.float32), mesh=mesh)
    def gather_kernel(tab_hbm, i_hbm, o_hbm):
        def body(i_vmem, o_vmem):
            pltpu.sync_copy(tab_hbm.at[i_vmem.at[0]], o_vmem)

        pltpu.emit_pipeline(
            body,
            grid=(n // _GATHER_WINDOW,),
            in_specs=[pl.BlockSpec((1, _GATHER_WINDOW), lambda i: (0, i))],
            out_specs=[pl.BlockSpec((_GATHER_WINDOW, width),
                                    lambda i: (i, 0))],
            core_axis_name=("core", "subcore"),
            dimension_semantics=(pltpu.PARALLEL,),
        )(i_hbm, o_hbm)

    return gather_kernel(table, indices)


def kernel(x, embeddings):
    n = x.shape[0] * x.shape[1]
    flat_x = x.reshape(n, _DIM)
    idx, acc = _distance_argmin(flat_x, embeddings)
    loss = acc[0, 0] * (1.25 / (n * _DIM))
    table = jnp.pad(embeddings.T, ((0, 0), (0, 128 - _DIM)))
    quantized = _sc_gather(table, idx.reshape(1, n))[:, :_DIM]
    return quantized.reshape(x.shape), loss


# speedup vs baseline: 1.1176x; 1.0046x over previous
import jax
import jax.numpy as jnp
from jax.experimental import pallas as pl
from jax.experimental.pallas import tpu as pltpu
from jax.experimental.pallas import tpu_sc as plsc

_DIM = 64
_NEMB = 1024
_ROWS_PER_BLOCK = 1024
_GATHER_WINDOW = 256


def _distance_argmin_body(x_ref, e_ref, idx_ref, acc_ref):
    i = pl.program_id(0)
    xb = x_ref[...]
    emb = e_ref[...]
    neg2m = jax.lax.dot_general(
        xb * -2.0, emb,
        dimension_numbers=(((1,), (0,)), ((), ())),
        preferred_element_type=jnp.float32,
    )
    x2 = jnp.sum(xb * xb, axis=1, keepdims=True)
    e2 = jnp.sum(emb * emb, axis=0, keepdims=True)
    d = (x2 + e2) + neg2m
    m = jnp.min(d, axis=1, keepdims=True)
    lane = jax.lax.broadcasted_iota(jnp.int32, d.shape, 1)
    idx = jnp.min(jnp.where(d == m, lane, jnp.int32(1 << 30)), axis=1)
    idx_ref[...] = idx.astype(jnp.int32)

    @pl.when(i == 0)
    def _():
        acc_ref[...] = jnp.zeros_like(acc_ref)

    acc_ref[...] += jnp.full(acc_ref.shape, jnp.sum(m), dtype=jnp.float32)


def _distance_argmin(flat_x, embeddings):
    n = flat_x.shape[0]
    nblk = n // _ROWS_PER_BLOCK
    return pl.pallas_call(
        _distance_argmin_body,
        grid=(nblk,),
        in_specs=[
            pl.BlockSpec((_ROWS_PER_BLOCK, _DIM), lambda i: (i, 0)),
            pl.BlockSpec((_DIM, _NEMB), lambda i: (0, 0)),
        ],
        out_specs=[
            pl.BlockSpec((_ROWS_PER_BLOCK,), lambda i: (i,)),
            pl.BlockSpec((8, 128), lambda i: (0, 0)),
        ],
        out_shape=[
            jax.ShapeDtypeStruct((n,), jnp.int32),
            jax.ShapeDtypeStruct((8, 128), jnp.float32),
        ],
        compiler_params=pltpu.CompilerParams(
            dimension_semantics=("arbitrary",)),
    )(flat_x, embeddings)


def _sc_gather(table, indices):
    n = indices.shape[1]
    width = table.shape[1]
    mesh = plsc.VectorSubcoreMesh(
        core_axis_name="core", subcore_axis_name="subcore")

    @pl.kernel(
        out_type=jax.ShapeDtypeStruct((n, width), jnp.float32), mesh=mesh)
    def gather_kernel(tab_hbm, i_hbm, o_hbm):
        def body(i_vmem, o_vmem):
            pltpu.sync_copy(tab_hbm.at[i_vmem.at[0]], o_vmem)

        pltpu.emit_pipeline(
            body,
            grid=(n // _GATHER_WINDOW,),
            in_specs=[pl.BlockSpec((1, _GATHER_WINDOW), lambda i: (0, i))],
            out_specs=[pl.BlockSpec((_GATHER_WINDOW, width),
                                    lambda i: (i, 0))],
            core_axis_name=("core", "subcore"),
            dimension_semantics=(pltpu.PARALLEL,),
        )(i_hbm, o_hbm)

    return gather_kernel(table, indices)


def kernel(x, embeddings):
    n = x.shape[0] * x.shape[1]
    flat_x = x.reshape(n, _DIM)
    idx, acc = _distance_argmin(flat_x, embeddings)
    loss = acc[0, 0] * (1.25 / (n * _DIM))
    table = jnp.pad(embeddings.T, ((0, 0), (0, 128 - _DIM)))
    quantized = _sc_gather(table, idx.reshape(1, n))[:, :_DIM]
    return quantized.reshape(x.shape), loss


# speedup vs baseline: 1.3304x; 1.1904x over previous
import jax
import jax.numpy as jnp
from jax.experimental import pallas as pl
from jax.experimental.pallas import tpu as pltpu
from jax.experimental.pallas import tpu_sc as plsc

_DIM = 64
_NEMB = 1024
_ROWS_PER_BLOCK = 1024
_GATHER_WINDOW = 256


def _distance_argmin_body(x_ref, e_ref, idx_ref, acc_ref):
    i = pl.program_id(0)
    xb = x_ref[...]
    emb = e_ref[...]
    neg2m = jax.lax.dot_general(
        xb * -2.0, emb,
        dimension_numbers=(((1,), (0,)), ((), ())),
        preferred_element_type=jnp.float32,
    )
    x2 = jnp.sum(xb * xb, axis=1, keepdims=True)
    e2 = jnp.sum(emb * emb, axis=0, keepdims=True)
    d = (x2 + e2) + neg2m
    m = jnp.min(d, axis=1, keepdims=True)
    lane = jax.lax.broadcasted_iota(jnp.int32, d.shape, 1)
    idx = jnp.min(jnp.where(d == m, lane, jnp.int32(1 << 30)), axis=1)
    idx_ref[...] = idx.astype(jnp.int32)

    @pl.when(i == 0)
    def _():
        acc_ref[...] = jnp.zeros_like(acc_ref)

    acc_ref[...] += jnp.full(acc_ref.shape, jnp.sum(m), dtype=jnp.float32)


def _distance_argmin(flat_x, embeddings):
    n = flat_x.shape[0]
    nblk = n // _ROWS_PER_BLOCK
    return pl.pallas_call(
        _distance_argmin_body,
        grid=(nblk,),
        in_specs=[
            pl.BlockSpec((_ROWS_PER_BLOCK, _DIM), lambda i: (i, 0)),
            pl.BlockSpec((_DIM, _NEMB), lambda i: (0, 0)),
        ],
        out_specs=[
            pl.BlockSpec((_ROWS_PER_BLOCK,), lambda i: (i,)),
            pl.BlockSpec((8, 128), lambda i: (0, 0)),
        ],
        out_shape=[
            jax.ShapeDtypeStruct((n,), jnp.int32),
            jax.ShapeDtypeStruct((8, 128), jnp.float32),
        ],
        compiler_params=pltpu.CompilerParams(
            dimension_semantics=("arbitrary",)),
    )(flat_x, embeddings)


def _sc_gather(table, indices):
    n = indices.shape[0]
    width = table.shape[1]
    mesh = plsc.VectorSubcoreMesh(
        core_axis_name="core", subcore_axis_name="subcore")
    num_workers = mesh.num_cores * mesh.num_subcores
    per_worker = n // num_workers

    @pl.kernel(
        out_type=jax.ShapeDtypeStruct((n, width), jnp.float32),
        mesh=mesh,
        scratch_types=[
            pltpu.VMEM((per_worker,), jnp.int32),
            pltpu.VMEM((per_worker, width), jnp.float32),
            pltpu.SemaphoreType.DMA,
        ],
        compiler_params=pltpu.CompilerParams(use_tc_tiling_on_sc=False),
    )
    def gather_kernel(tab_hbm, i_hbm, o_hbm, idx_v, rows_v, sem):
        wid = (jax.lax.axis_index("subcore") * mesh.num_cores
               + jax.lax.axis_index("core"))
        base = wid * per_worker
        pltpu.sync_copy(i_hbm.at[pl.ds(base, per_worker)], idx_v)
        pltpu.async_copy(tab_hbm.at[idx_v], rows_v, sem).wait()
        pltpu.sync_copy(rows_v, o_hbm.at[pl.ds(base, per_worker)])

    return gather_kernel(table, indices)


def kernel(x, embeddings):
    n = x.shape[0] * x.shape[1]
    flat_x = x.reshape(n, _DIM)
    idx, acc = _distance_argmin(flat_x, embeddings)
    loss = acc[0, 0] * (1.25 / (n * _DIM))
    quantized = _sc_gather(embeddings.T, idx)
    return quantized.reshape(x.shape), loss
